# Initial kernel scaffold; baseline (speedup 1.0000x reference)
import jax
import jax.numpy as jnp
from jax.experimental import pallas as pl


def kernel(inputs, edge_index, W0, W1, gamma0, beta0):
    raise NotImplementedError("write your pallas kernel here")


# speedup vs baseline: 9.3136x; 9.3136x over previous
import jax
import jax.numpy as jnp
from jax import lax
from jax.experimental import pallas as pl
from jax.experimental.pallas import tpu as pltpu
from jax.experimental.pallas import tpu_sc as plsc

N = 10000
E = 320000
D = 128
EPS = 1e-5

NC = 2
NS = 16
NW = NC * NS
CHUNK = 125
EPW = E // NW
CPW = EPW // CHUNK
HN = 10240

_SC_MESH = plsc.VectorSubcoreMesh(
    core_axis_name="c", subcore_axis_name="s", num_cores=NC, num_subcores=NS)


def _sc_deg_body(dst_hbm, degp_hbm, dst_v, hist_v, sem):
    del sem
    c = lax.axis_index("c")
    s = lax.axis_index("s")
    w = s * NC + c

    pltpu.sync_copy(dst_hbm.at[w], dst_v)

    zero16 = jnp.zeros((16,), jnp.float32)

    @pl.loop(0, HN // 16)
    def _z(i):
        hist_v[pl.ds(i * 16, 16)] = zero16

    one16 = jnp.ones((16,), jnp.float32)

    @pl.loop(0, EPW // 16)
    def _h(i):
        idx = dst_v[pl.ds(i * 16, 16)]
        plsc.addupdate_scatter(hist_v, [idx], one16)

    pltpu.sync_copy(hist_v, degp_hbm.at[w])


_sc_deg = pl.kernel(
    _sc_deg_body,
    out_type=jax.ShapeDtypeStruct((NW, HN), jnp.float32),
    mesh=_SC_MESH,
    compiler_params=pltpu.CompilerParams(needs_layout_passes=False),
    scratch_types=[
        pltpu.VMEM((EPW,), jnp.int32),
        pltpu.VMEM((HN,), jnp.float32),
        pltpu.SemaphoreType.DMA,
    ],
    name="sc_gcn_degree",
)


def _sc_agg_body(h_hbm, src_hbm, dst_hbm, zeros_hbm, part_hbm,
                 src_v, dst_v, gbuf, acc, sem):
    c = lax.axis_index("c")
    s = lax.axis_index("s")
    w = s * NC + c

    pltpu.sync_copy(src_hbm.at[w], src_v)
    pltpu.sync_copy(dst_hbm.at[w], dst_v)

    @pl.when(s == 0)
    def _zero():
        pltpu.sync_copy(zeros_hbm, acc)

    plsc.subcore_barrier()

    @pl.loop(0, CPW)
    def _chunk(j):
        pltpu.async_copy(h_hbm.at[src_v.at[j]], gbuf, sem).wait()
        pltpu.sync_copy(gbuf, acc.at[dst_v.at[j]], add=True)

    plsc.subcore_barrier()

    @pl.when(s == 0)
    def _copy_out():
        pltpu.sync_copy(acc, part_hbm.at[c])


_sc_aggregate = pl.kernel(
    _sc_agg_body,
    out_type=jax.ShapeDtypeStruct((NC, N, D), jnp.float32),
    mesh=_SC_MESH,
    scratch_types=[
        pltpu.VMEM((CPW, CHUNK), jnp.int32),
        pltpu.VMEM((CPW, CHUNK), jnp.int32),
        pltpu.VMEM((CHUNK, D), jnp.float32),
        pltpu.MemorySpace.VMEM_SHARED((N, D), jnp.float32),
        pltpu.SemaphoreType.DMA,
    ],
    name="sc_gcn_aggregate",
)


def _tc1_body(p_ref, dt_ref, w0_ref, g_ref, b_ref, h1_ref):
    deg = jnp.sum(dt_ref[...], axis=1, keepdims=True)
    t = (p_ref[0] + p_ref[1]) / jnp.maximum(deg, 1.0)
    z = jnp.dot(t, w0_ref[...], preferred_element_type=jnp.float32)
    mean = jnp.mean(z, axis=0, keepdims=True)
    var = jnp.mean((z - mean) * (z - mean), axis=0, keepdims=True)
    zn = g_ref[...] * (z - mean) / jnp.sqrt(var + EPS) + b_ref[...]
    h1_ref[...] = jnp.maximum(zn, 0.0)


def _tc2_body(q_ref, dt_ref, w1_ref, o_ref):
    deg = jnp.sum(dt_ref[...], axis=1, keepdims=True)
    t = (q_ref[0] + q_ref[1]) / jnp.maximum(deg, 1.0)
    z = jnp.dot(t, w1_ref[...], preferred_element_type=jnp.float32)
    o_ref[...] = jnp.maximum(z, 0.0)


_tc1 = pl.pallas_call(
    _tc1_body,
    out_shape=jax.ShapeDtypeStruct((N, D), jnp.float32),
    name="tc_gcn_norm_matmul",
)

_tc2 = pl.pallas_call(
    _tc2_body,
    out_shape=jax.ShapeDtypeStruct((N, D), jnp.float32),
    name="tc_gcn_matmul_out",
)


@jax.jit
def kernel(inputs, edge_index, W0, W1, gamma0, beta0):
    src = edge_index[0].astype(jnp.int32).reshape(NW, CPW, CHUNK)
    dst = edge_index[1].astype(jnp.int32).reshape(NW, CPW, CHUNK)
    dst2 = edge_index[1].astype(jnp.int32).reshape(NW, EPW)
    zeros_nd = jnp.zeros((N, D), jnp.float32)
    gamma = gamma0.reshape(1, D)
    beta = beta0.reshape(1, D)

    degp = _sc_deg(dst2)
    degt = degp.T[:N]
    part = _sc_aggregate(inputs, src, dst, zeros_nd)
    h1 = _tc1(part, degt, W0, gamma, beta)
    qart = _sc_aggregate(h1, src, dst, zeros_nd)
    return _tc2(qart, degt, W1)


# speedup vs baseline: 13.7304x; 1.4742x over previous
import jax
import jax.numpy as jnp
from jax import lax
from jax.experimental import pallas as pl
from jax.experimental.pallas import tpu as pltpu
from jax.experimental.pallas import tpu_sc as plsc

N = 10000
E = 320000
D = 128
EPS = 1e-5

NC = 2
NS = 16
NW = NC * NS
CHUNK = 125
EPW = E // NW
CPW = EPW // CHUNK
HN = 10240

_SC_MESH = plsc.VectorSubcoreMesh(
    core_axis_name="c", subcore_axis_name="s", num_cores=NC, num_subcores=NS)


def _sc_deg_body(dst_hbm, degp_hbm, dst_v, hist_v, sem):
    del sem
    c = lax.axis_index("c")
    s = lax.axis_index("s")
    w = s * NC + c

    pltpu.sync_copy(dst_hbm.at[w], dst_v)

    zero16 = jnp.zeros((16,), jnp.float32)

    @pl.loop(0, HN // 16)
    def _z(i):
        hist_v[pl.ds(i * 16, 16)] = zero16

    one16 = jnp.ones((16,), jnp.float32)

    @pl.loop(0, EPW // 16)
    def _h(i):
        idx = dst_v[pl.ds(i * 16, 16)]
        plsc.addupdate_scatter(hist_v, [idx], one16)

    pltpu.sync_copy(hist_v, degp_hbm.at[w])


_sc_deg = pl.kernel(
    _sc_deg_body,
    out_type=jax.ShapeDtypeStruct((NW, HN), jnp.float32),
    mesh=_SC_MESH,
    compiler_params=pltpu.CompilerParams(needs_layout_passes=False),
    scratch_types=[
        pltpu.VMEM((EPW,), jnp.int32),
        pltpu.VMEM((HN,), jnp.float32),
        pltpu.SemaphoreType.DMA,
    ],
    name="sc_gcn_degree",
)


BROWS = 8
NBLK = CPW // BROWS


def _sc_agg_body(h_hbm, src_hbm, dst_hbm, zeros_hbm, part_hbm,
                 src_v0, src_v1, dst_v0, dst_v1, gbuf0, gbuf1, acc,
                 sem_g0, sem_g1, sem_i0, sem_i1):
    c = lax.axis_index("c")
    s = lax.axis_index("s")
    w = s * NC + c

    src_slots = (src_v0, src_v1)
    dst_slots = (dst_v0, dst_v1)
    gbufs = (gbuf0, gbuf1)
    gsems = (sem_g0, sem_g1)
    isems = (sem_i0, sem_i1)

    def idx_block(hbm, b):
        return hbm.at[w, pl.ds(b * BROWS, BROWS)]

    def issue_idx(b):
        sl = b & 1
        pltpu.async_copy(idx_block(src_hbm, b), src_slots[sl], isems[sl])
        pltpu.async_copy(idx_block(dst_hbm, b), dst_slots[sl], isems[sl])

    def wait_idx(b):
        sl = b & 1
        pltpu.make_async_copy(idx_block(src_hbm, b), src_slots[sl],
                              isems[sl]).wait()
        pltpu.make_async_copy(idx_block(dst_hbm, b), dst_slots[sl],
                              isems[sl]).wait()

    def gather_desc(j):
        sl = j & 1
        src_row = src_slots[(j // BROWS) & 1].at[j % BROWS]
        return pltpu.make_async_copy(h_hbm.at[src_row], gbufs[sl], gsems[sl])

    @pl.when(s == 0)
    def _zero():
        pltpu.sync_copy(zeros_hbm, acc)

    issue_idx(0)
    wait_idx(0)
    issue_idx(1)
    plsc.subcore_barrier()
    gather_desc(0).start()
    gather_desc(1).start()

    for j in range(CPW):
        blk, j2 = divmod(j, BROWS)
        gather_desc(j).wait()
        pltpu.sync_copy(gbufs[j & 1], acc.at[dst_slots[blk & 1].at[j2]],
                        add=True)
        if j + 2 < CPW:
            nb = (j + 2) // BROWS
            if (j + 2) % BROWS == 0:
                wait_idx(nb)
            gather_desc(j + 2).start()
        if j2 == BROWS - 1 and blk + 2 < NBLK:
            issue_idx(blk + 2)

    plsc.subcore_barrier()

    @pl.when(s == 0)
    def _copy_out():
        pltpu.sync_copy(acc, part_hbm.at[c])


_sc_aggregate = pl.kernel(
    _sc_agg_body,
    out_type=jax.ShapeDtypeStruct((NC, N, D), jnp.float32),
    mesh=_SC_MESH,
    scratch_types=[
        pltpu.VMEM((BROWS, CHUNK), jnp.int32),
        pltpu.VMEM((BROWS, CHUNK), jnp.int32),
        pltpu.VMEM((BROWS, CHUNK), jnp.int32),
        pltpu.VMEM((BROWS, CHUNK), jnp.int32),
        pltpu.VMEM((CHUNK, D), jnp.float32),
        pltpu.VMEM((CHUNK, D), jnp.float32),
        pltpu.MemorySpace.VMEM_SHARED((N, D), jnp.float32),
        pltpu.SemaphoreType.DMA,
        pltpu.SemaphoreType.DMA,
        pltpu.SemaphoreType.DMA,
        pltpu.SemaphoreType.DMA,
    ],
    name="sc_gcn_aggregate",
)


def _tc1_body(p_ref, dt_ref, w0_ref, g_ref, b_ref, h1_ref):
    deg = jnp.sum(dt_ref[...], axis=1, keepdims=True)
    t = (p_ref[0] + p_ref[1]) / jnp.maximum(deg, 1.0)
    z = jnp.dot(t, w0_ref[...], preferred_element_type=jnp.float32)
    mean = jnp.mean(z, axis=0, keepdims=True)
    var = jnp.mean((z - mean) * (z - mean), axis=0, keepdims=True)
    zn = g_ref[...] * (z - mean) / jnp.sqrt(var + EPS) + b_ref[...]
    h1_ref[...] = jnp.maximum(zn, 0.0)


def _tc2_body(q_ref, dt_ref, w1_ref, o_ref):
    deg = jnp.sum(dt_ref[...], axis=1, keepdims=True)
    t = (q_ref[0] + q_ref[1]) / jnp.maximum(deg, 1.0)
    z = jnp.dot(t, w1_ref[...], preferred_element_type=jnp.float32)
    o_ref[...] = jnp.maximum(z, 0.0)


_tc1 = pl.pallas_call(
    _tc1_body,
    out_shape=jax.ShapeDtypeStruct((N, D), jnp.float32),
    name="tc_gcn_norm_matmul",
)

_tc2 = pl.pallas_call(
    _tc2_body,
    out_shape=jax.ShapeDtypeStruct((N, D), jnp.float32),
    name="tc_gcn_matmul_out",
)


@jax.jit
def kernel(inputs, edge_index, W0, W1, gamma0, beta0):
    src = edge_index[0].astype(jnp.int32).reshape(NW, CPW, CHUNK)
    dst = edge_index[1].astype(jnp.int32).reshape(NW, CPW, CHUNK)
    dst2 = edge_index[1].astype(jnp.int32).reshape(NW, EPW)
    zeros_nd = jnp.zeros((N, D), jnp.float32)
    gamma = gamma0.reshape(1, D)
    beta = beta0.reshape(1, D)

    degp = _sc_deg(dst2)
    degt = degp.T[:N]
    part = _sc_aggregate(inputs, src, dst, zeros_nd)
    h1 = _tc1(part, degt, W0, gamma, beta)
    qart = _sc_aggregate(h1, src, dst, zeros_nd)
    return _tc2(qart, degt, W1)


# speedup vs baseline: 14.0003x; 1.0197x over previous
import jax
import jax.numpy as jnp
from jax import lax
from jax.experimental import pallas as pl
from jax.experimental.pallas import tpu as pltpu
from jax.experimental.pallas import tpu_sc as plsc

N = 10000
E = 320000
D = 128
EPS = 1e-5

NC = 2
NS = 16
NW = NC * NS
CHUNK = 125
EPW = E // NW
CPW = EPW // CHUNK
HN = 10240

_SC_MESH = plsc.VectorSubcoreMesh(
    core_axis_name="c", subcore_axis_name="s", num_cores=NC, num_subcores=NS)


def _sc_deg_body(dst_hbm, degp_hbm, dst_v, hist_v, sem):
    del sem
    c = lax.axis_index("c")
    s = lax.axis_index("s")
    w = s * NC + c

    pltpu.sync_copy(dst_hbm.at[w], dst_v)

    zero16 = jnp.zeros((16,), jnp.float32)

    @pl.loop(0, HN // 16)
    def _z(i):
        hist_v[pl.ds(i * 16, 16)] = zero16

    one16 = jnp.ones((16,), jnp.float32)

    @pl.loop(0, EPW // 16)
    def _h(i):
        idx = dst_v[pl.ds(i * 16, 16)]
        plsc.addupdate_scatter(hist_v, [idx], one16)

    pltpu.sync_copy(hist_v, degp_hbm.at[w])


_sc_deg = pl.kernel(
    _sc_deg_body,
    out_type=jax.ShapeDtypeStruct((NW, HN), jnp.float32),
    mesh=_SC_MESH,
    compiler_params=pltpu.CompilerParams(needs_layout_passes=False),
    scratch_types=[
        pltpu.VMEM((EPW,), jnp.int32),
        pltpu.VMEM((HN,), jnp.float32),
        pltpu.SemaphoreType.DMA,
    ],
    name="sc_gcn_degree",
)


BROWS = 8
NBLK = CPW // BROWS


def _sc_agg_body(h_hbm, src_hbm, dst_hbm, zeros_hbm, part_hbm,
                 src_v0, src_v1, dst_v0, dst_v1, gbuf0, gbuf1, acc,
                 sem_g0, sem_g1, sem_i0, sem_i1, sem_z):
    c = lax.axis_index("c")
    s = lax.axis_index("s")
    w = s * NC + c

    src_slots = (src_v0, src_v1)
    dst_slots = (dst_v0, dst_v1)
    gbufs = (gbuf0, gbuf1)
    gsems = (sem_g0, sem_g1)
    isems = (sem_i0, sem_i1)

    def idx_block(hbm, b):
        return hbm.at[w, pl.ds(b * BROWS, BROWS)]

    def issue_idx(b):
        sl = b & 1
        pltpu.async_copy(idx_block(src_hbm, b), src_slots[sl], isems[sl])
        pltpu.async_copy(idx_block(dst_hbm, b), dst_slots[sl], isems[sl])

    def wait_idx(b):
        sl = b & 1
        pltpu.make_async_copy(idx_block(src_hbm, b), src_slots[sl],
                              isems[sl]).wait()
        pltpu.make_async_copy(idx_block(dst_hbm, b), dst_slots[sl],
                              isems[sl]).wait()

    def gather_desc(j):
        sl = j & 1
        src_row = src_slots[(j // BROWS) & 1].at[j % BROWS]
        return pltpu.make_async_copy(h_hbm.at[src_row], gbufs[sl], gsems[sl])

    def zero_desc():
        if_last = s == NS - 1
        lo = pltpu.make_async_copy(zeros_hbm.at[pl.ds(s * 624, 624)],
                                  acc.at[pl.ds(s * 624, 624)], sem_z)
        hi = pltpu.make_async_copy(zeros_hbm.at[pl.ds(N - 640, 640)],
                                  acc.at[pl.ds(N - 640, 640)], sem_z)
        return if_last, lo, hi

    is_last, zlo, zhi = zero_desc()

    @pl.when(is_last)
    def _z_hi():
        zhi.start()

    @pl.when(jnp.logical_not(is_last))
    def _z_lo():
        zlo.start()

    issue_idx(0)
    wait_idx(0)
    issue_idx(1)
    gather_desc(0).start()
    gather_desc(1).start()

    @pl.when(is_last)
    def _zw_hi():
        zhi.wait()

    @pl.when(jnp.logical_not(is_last))
    def _zw_lo():
        zlo.wait()

    plsc.subcore_barrier()

    for j in range(CPW):
        blk, j2 = divmod(j, BROWS)
        gather_desc(j).wait()
        pltpu.sync_copy(gbufs[j & 1], acc.at[dst_slots[blk & 1].at[j2]],
                        add=True)
        if j + 2 < CPW:
            nb = (j + 2) // BROWS
            if (j + 2) % BROWS == 0:
                wait_idx(nb)
            gather_desc(j + 2).start()
        if j2 == BROWS - 1 and blk + 2 < NBLK:
            issue_idx(blk + 2)

    plsc.subcore_barrier()

    @pl.when(is_last)
    def _co_hi():
        pltpu.sync_copy(acc.at[pl.ds(N - 640, 640)],
                        part_hbm.at[c, pl.ds(N - 640, 640)])

    @pl.when(jnp.logical_not(is_last))
    def _co_lo():
        pltpu.sync_copy(acc.at[pl.ds(s * 624, 624)],
                        part_hbm.at[c, pl.ds(s * 624, 624)])


_sc_aggregate = pl.kernel(
    _sc_agg_body,
    out_type=jax.ShapeDtypeStruct((NC, N, D), jnp.float32),
    mesh=_SC_MESH,
    scratch_types=[
        pltpu.VMEM((BROWS, CHUNK), jnp.int32),
        pltpu.VMEM((BROWS, CHUNK), jnp.int32),
        pltpu.VMEM((BROWS, CHUNK), jnp.int32),
        pltpu.VMEM((BROWS, CHUNK), jnp.int32),
        pltpu.VMEM((CHUNK, D), jnp.float32),
        pltpu.VMEM((CHUNK, D), jnp.float32),
        pltpu.MemorySpace.VMEM_SHARED((N, D), jnp.float32),
        pltpu.SemaphoreType.DMA,
        pltpu.SemaphoreType.DMA,
        pltpu.SemaphoreType.DMA,
        pltpu.SemaphoreType.DMA,
        pltpu.SemaphoreType.DMA,
    ],
    name="sc_gcn_aggregate",
)


def _tc1_body(p_ref, dt_ref, w0_ref, g_ref, b_ref, h1_ref):
    deg = jnp.sum(dt_ref[...], axis=1, keepdims=True)
    t = (p_ref[0] + p_ref[1]) / jnp.maximum(deg, 1.0)
    z = jnp.dot(t, w0_ref[...], preferred_element_type=jnp.float32)
    mean = jnp.mean(z, axis=0, keepdims=True)
    var = jnp.mean((z - mean) * (z - mean), axis=0, keepdims=True)
    zn = g_ref[...] * (z - mean) / jnp.sqrt(var + EPS) + b_ref[...]
    h1_ref[...] = jnp.maximum(zn, 0.0)


def _tc2_body(q_ref, dt_ref, w1_ref, o_ref):
    deg = jnp.sum(dt_ref[...], axis=1, keepdims=True)
    t = (q_ref[0] + q_ref[1]) / jnp.maximum(deg, 1.0)
    z = jnp.dot(t, w1_ref[...], preferred_element_type=jnp.float32)
    o_ref[...] = jnp.maximum(z, 0.0)


_tc1 = pl.pallas_call(
    _tc1_body,
    out_shape=jax.ShapeDtypeStruct((N, D), jnp.float32),
    name="tc_gcn_norm_matmul",
)

_tc2 = pl.pallas_call(
    _tc2_body,
    out_shape=jax.ShapeDtypeStruct((N, D), jnp.float32),
    name="tc_gcn_matmul_out",
)


@jax.jit
def kernel(inputs, edge_index, W0, W1, gamma0, beta0):
    src = edge_index[0].astype(jnp.int32).reshape(NW, CPW, CHUNK)
    dst = edge_index[1].astype(jnp.int32).reshape(NW, CPW, CHUNK)
    dst2 = edge_index[1].astype(jnp.int32).reshape(NW, EPW)
    zeros_nd = jnp.zeros((N, D), jnp.float32)
    gamma = gamma0.reshape(1, D)
    beta = beta0.reshape(1, D)

    degp = _sc_deg(dst2)
    degt = degp.T[:N]
    part = _sc_aggregate(inputs, src, dst, zeros_nd)
    h1 = _tc1(part, degt, W0, gamma, beta)
    qart = _sc_aggregate(h1, src, dst, zeros_nd)
    return _tc2(qart, degt, W1)
